# Initial kernel scaffold; baseline (speedup 1.0000x reference)
import jax
import jax.numpy as jnp
from jax.experimental import pallas as pl


def kernel(classifications, regressions, annotations):
    raise NotImplementedError("write your pallas kernel here")


# speedup vs baseline: 1.4558x; 1.4558x over previous
import functools

import jax
import jax.numpy as jnp
from jax import lax
from jax.experimental import pallas as pl
from jax.experimental.pallas import tpu as pltpu
from jax.experimental.pallas import tpu_sc as plsc

_B, _N, _C, _A = 16, 8192, 80, 64
_NV = 48
_RB = 1024
_K = _N // _RB


def _sc_gather(ann_t, reg2d):
    mesh = plsc.VectorSubcoreMesh(core_axis_name="c", subcore_axis_name="s")

    @functools.partial(
        pl.kernel,
        mesh=mesh,
        out_type=jax.ShapeDtypeStruct((_B, _NV, _C), jnp.float32),
        scratch_types=[
            pltpu.VMEM((_A,), jnp.float32),
            pltpu.VMEM((_A,), jnp.float32),
            pltpu.VMEM((_NV,), jnp.int32),
            pltpu.VMEM((_NV, _C), jnp.float32),
            pltpu.SemaphoreType.DMA,
        ],
        compiler_params=pltpu.CompilerParams(use_tc_tiling_on_sc=False),
    )
    def k(ann_hbm, reg_hbm, out_hbm, s_v, e_v, idx_v, rows_v, sem):
        cid = lax.axis_index("c")
        sid = lax.axis_index("s")
        wid = sid * 2 + cid

        @pl.when(wid < _B)
        def _():
            b = wid
            pltpu.sync_copy(ann_hbm.at[b, 0], s_v)
            pltpu.sync_copy(ann_hbm.at[b, 1], e_v)
            for j in range(_NV // 16):
                sv = s_v[pl.ds(j * 16, 16)]
                ev = e_v[pl.ds(j * 16, 16)]
                tp = ((sv + ev) * 0.5) * 100.0
                idx_v[pl.ds(j * 16, 16)] = tp.astype(jnp.int32) + b * _N
            pltpu.async_copy(reg_hbm.at[idx_v], rows_v, sem).wait()
            pltpu.sync_copy(rows_v, out_hbm.at[b])

    return k(ann_t, reg2d)


def _smooth_l1(d):
    return jnp.where(d <= 1.0, 0.5 * d * d, d - 0.5)


def _tc_body(cls_ref, ann_ref, rrows_ref, out_cls_ref, out_reg_ref, acc_ref, cg_ref):
    b = pl.program_id(0)
    k = pl.program_id(1)

    @pl.when((b == 0) & (k == 0))
    def _init():
        acc_ref[0] = 0.0
        acc_ref[1] = 0.0

    xc = jnp.clip(cls_ref[0], 0.0001, 1.0 - 0.0001)
    part = jnp.sum(xc * xc * jnp.log(1.0 - xc))
    acc_ref[0] = acc_ref[0] + (-0.75) * part

    ann = ann_ref[0]
    s = ann[0:1, :]
    e = ann[1:2, :]
    cl = ann[2:3, :]
    pii = (((s + e) * 0.5) * 100.0).astype(jnp.int32)
    pif = pii.astype(jnp.float32)

    ident = (lax.broadcasted_iota(jnp.int32, (_A, _A), 0)
             == lax.broadcasted_iota(jnp.int32, (_A, _A), 1)).astype(jnp.float32)

    def to_sub(v):
        return jnp.sum(ident * v, axis=1, keepdims=True)

    pi_s = to_sub(pif)
    pi_si = pi_s[:_NV, :].astype(jnp.int32)

    rowid = lax.broadcasted_iota(jnp.int32, (_NV, _RB), 1) + k * _RB
    oh = (rowid == pi_si).astype(jnp.float32)
    contrib = jnp.dot(oh, xc, preferred_element_type=jnp.float32,
                      precision=lax.Precision.HIGHEST)

    @pl.when(k == 0)
    def _set():
        cg_ref[...] = contrib

    @pl.when(k > 0)
    def _add():
        cg_ref[...] = cg_ref[...] + contrib

    @pl.when(k == _K - 1)
    def _corr():
        clf = cl.astype(jnp.int32).astype(jnp.float32)
        lane = lax.broadcasted_iota(jnp.int32, (1, _A), 1)
        validl = (lane < _NV).astype(jnp.float32)
        keyl = pif * float(_C) + clf

        key_s = to_sub(keyl)
        valid_s = to_sub(validl)
        js = lax.broadcasted_iota(jnp.int32, (_A, _A), 0)
        ks = lax.broadcasted_iota(jnp.int32, (_A, _A), 1)
        dup = (key_s == keyl) & (js < ks) & (valid_s > 0.0)
        dup_l = jnp.sum(dup.astype(jnp.float32), axis=0, keepdims=True)
        m_l = validl * (dup_l == 0.0).astype(jnp.float32)
        cntm = (pi_s == pif) & (valid_s > 0.0)
        cnt_l = jnp.sum(cntm.astype(jnp.float32), axis=0, keepdims=True)

        m_s = to_sub(m_l)[:_NV, :]
        cnt_s = to_sub(cnt_l)[:_NV, :]
        cl_s = to_sub(clf)[:_NV, :].astype(jnp.int32)

        lane80 = lax.broadcasted_iota(jnp.int32, (_NV, _C), 1)
        onehot = lane80 == cl_s
        cg = jnp.sum(jnp.where(onehot, cg_ref[...], 0.0), axis=1, keepdims=True)
        rrows = rrows_ref[0]
        g = jnp.sum(jnp.where(onehot, rrows, 0.0), axis=1, keepdims=True)

        l1 = -0.25 * (1.0 - cg) * (1.0 - cg) * jnp.log(cg)
        l0 = -0.75 * cg * cg * jnp.log(1.0 - cg)
        cls_corr = jnp.sum(m_s * (l1 - l0))

        reg_base = jnp.sum(_smooth_l1(jnp.abs(rrows)))
        reg_corr = jnp.sum(m_s * cnt_s * (_smooth_l1(jnp.abs(1.0 - g))
                                          - _smooth_l1(jnp.abs(g))))

        acc_ref[0] = acc_ref[0] + cls_corr
        acc_ref[1] = acc_ref[1] + reg_base + reg_corr

    @pl.when((b == _B - 1) & (k == _K - 1))
    def _fin():
        out_cls_ref[0, 0] = acc_ref[0] / float(_NV * _B)
        out_reg_ref[0, 0] = acc_ref[1] / float(_NV * _C * _B)


def _tc_call_kwargs():
    return dict(
        grid=(_B, _K),
        in_specs=[
            pl.BlockSpec((1, _RB, _C), lambda b, k: (b, k, 0)),
            pl.BlockSpec((1, 3, _A), lambda b, k: (b, 0, 0)),
            pl.BlockSpec((1, _NV, _C), lambda b, k: (b, 0, 0)),
        ],
        out_specs=[
            pl.BlockSpec(memory_space=pltpu.SMEM),
            pl.BlockSpec(memory_space=pltpu.SMEM),
        ],
        out_shape=[
            jax.ShapeDtypeStruct((1, 1), jnp.float32),
            jax.ShapeDtypeStruct((1, 1), jnp.float32),
        ],
        scratch_shapes=[
            pltpu.SMEM((2,), jnp.float32),
            pltpu.VMEM((_NV, _C), jnp.float32),
        ],
    )


def kernel(classifications, regressions, annotations):
    reg2d = regressions.reshape(_B * _N, _C)
    ann_t = annotations.transpose(0, 2, 1)
    rrows = _sc_gather(ann_t, reg2d)

    out_c, out_r = pl.pallas_call(_tc_body, **_tc_call_kwargs())(
        classifications, ann_t, rrows)
    return out_c.reshape(1), out_r.reshape(1)


# speedup vs baseline: 1.6603x; 1.1405x over previous
import functools

import jax
import jax.numpy as jnp
from jax import lax
from jax.experimental import pallas as pl
from jax.experimental.pallas import tpu as pltpu
from jax.experimental.pallas import tpu_sc as plsc

_B, _N, _C, _A = 16, 8192, 80, 64
_NV = 48


def _sc_gather(ann_t, reg2d):
    mesh = plsc.VectorSubcoreMesh(core_axis_name="c", subcore_axis_name="s")

    @functools.partial(
        pl.kernel,
        mesh=mesh,
        out_type=jax.ShapeDtypeStruct((_B, _NV, _C), jnp.float32),
        scratch_types=[
            pltpu.VMEM((_A,), jnp.float32),
            pltpu.VMEM((_A,), jnp.float32),
            pltpu.VMEM((_NV,), jnp.int32),
            pltpu.VMEM((_NV, _C), jnp.float32),
            pltpu.SemaphoreType.DMA,
        ],
        compiler_params=pltpu.CompilerParams(use_tc_tiling_on_sc=False),
    )
    def k(ann_hbm, reg_hbm, out_hbm, s_v, e_v, idx_v, rows_v, sem):
        cid = lax.axis_index("c")
        sid = lax.axis_index("s")
        wid = sid * 2 + cid

        @pl.when(wid < _B)
        def _():
            b = wid
            pltpu.sync_copy(ann_hbm.at[b, 0], s_v)
            pltpu.sync_copy(ann_hbm.at[b, 1], e_v)
            for j in range(_NV // 16):
                sv = s_v[pl.ds(j * 16, 16)]
                ev = e_v[pl.ds(j * 16, 16)]
                tp = ((sv + ev) * 0.5) * 100.0
                idx_v[pl.ds(j * 16, 16)] = tp.astype(jnp.int32) + b * _N
            pltpu.async_copy(reg_hbm.at[idx_v], rows_v, sem).wait()
            pltpu.sync_copy(rows_v, out_hbm.at[b])

    return k(ann_t, reg2d)


def _ident64():
    return (lax.broadcasted_iota(jnp.int32, (_A, _A), 0)
            == lax.broadcasted_iota(jnp.int32, (_A, _A), 1)).astype(jnp.float32)


def _pi_lanes(ann):
    s = ann[0:1, :]
    e = ann[1:2, :]
    return (((s + e) * 0.5) * 100.0).astype(jnp.int32)


def _smooth_l1(d):
    return jnp.where(d <= 1.0, 0.5 * d * d, d - 0.5)


def _dense_body(cls_ref, ann_ref, tot_ref, cg_ref, acc_ref):
    b = pl.program_id(0)

    xc = jnp.clip(cls_ref[0], 0.0001, 1.0 - 0.0001)
    part = jnp.sum(xc * xc * jnp.log(1.0 - xc))

    @pl.when(b == 0)
    def _init():
        acc_ref[0] = 0.0

    acc_ref[0] = acc_ref[0] + (-0.75) * part

    ident = _ident64()
    pif = _pi_lanes(ann_ref[0]).astype(jnp.float32)
    pi_s = jnp.sum(ident * pif, axis=1, keepdims=True)
    pi_si = pi_s[:_NV, :].astype(jnp.int32)

    rowid = lax.broadcasted_iota(jnp.int32, (_NV, _N), 1)
    oh = (rowid == pi_si).astype(jnp.float32)
    contrib = jnp.dot(oh, xc, preferred_element_type=jnp.float32,
                      precision=lax.Precision.HIGHEST)

    clf = ann_ref[0][2:3, :].astype(jnp.int32).astype(jnp.float32)
    cl_s = jnp.sum(ident * clf, axis=1, keepdims=True)[:_NV, :].astype(jnp.int32)
    lane80 = lax.broadcasted_iota(jnp.int32, (_NV, _C), 1)
    cg_s = jnp.sum(jnp.where(lane80 == cl_s, contrib, 0.0),
                   axis=1, keepdims=True)
    cg_l = jnp.sum(ident[:_NV, :_NV] * cg_s, axis=0, keepdims=True)
    cg_ref[0] = cg_l

    @pl.when(b == _B - 1)
    def _fin():
        tot_ref[0, 0] = acc_ref[0]


def _final_body(ann_ref, rrows_ref, cg_ref, tot_ref, out_cls_ref, out_reg_ref,
                acc_ref):
    b = pl.program_id(0)

    @pl.when(b == 0)
    def _init():
        acc_ref[0] = 0.0
        acc_ref[1] = 0.0

    ann = ann_ref[0]
    cl = ann[2:3, :]
    pii = _pi_lanes(ann)
    pif = pii.astype(jnp.float32)
    clf = cl.astype(jnp.int32).astype(jnp.float32)
    lane = lax.broadcasted_iota(jnp.int32, (1, _A), 1)
    validl = (lane < _NV).astype(jnp.float32)
    keyl = pif * float(_C) + clf

    ident = _ident64()

    def to_sub(v):
        return jnp.sum(ident * v, axis=1, keepdims=True)

    key_s = to_sub(keyl)
    pi_s = to_sub(pif)
    valid_s = to_sub(validl)
    js = lax.broadcasted_iota(jnp.int32, (_A, _A), 0)
    ks = lax.broadcasted_iota(jnp.int32, (_A, _A), 1)
    dup = (key_s == keyl) & (js < ks) & (valid_s > 0.0)
    dup_l = jnp.sum(dup.astype(jnp.float32), axis=0, keepdims=True)
    m_l = validl * (dup_l == 0.0).astype(jnp.float32)
    cntm = (pi_s == pif) & (valid_s > 0.0)
    cnt_l = jnp.sum(cntm.astype(jnp.float32), axis=0, keepdims=True)

    m_s = to_sub(m_l)[:_NV, :]
    cnt_s = to_sub(cnt_l)[:_NV, :]
    cl_s = to_sub(clf)[:_NV, :].astype(jnp.int32)

    cgl = cg_ref[0]
    cg = jnp.sum(ident[:_NV, :_NV] * cgl, axis=1, keepdims=True)

    rrows = rrows_ref[0]
    lane80 = lax.broadcasted_iota(jnp.int32, (_NV, _C), 1)
    onehot = lane80 == cl_s
    g = jnp.sum(jnp.where(onehot, rrows, 0.0), axis=1, keepdims=True)

    l1 = -0.25 * (1.0 - cg) * (1.0 - cg) * jnp.log(cg)
    l0 = -0.75 * cg * cg * jnp.log(1.0 - cg)
    cls_corr = jnp.sum(m_s * (l1 - l0))

    reg_base = jnp.sum(_smooth_l1(jnp.abs(rrows)))
    reg_corr = jnp.sum(m_s * cnt_s * (_smooth_l1(jnp.abs(1.0 - g))
                                      - _smooth_l1(jnp.abs(g))))

    acc_ref[0] = acc_ref[0] + cls_corr
    acc_ref[1] = acc_ref[1] + reg_base + reg_corr

    @pl.when(b == _B - 1)
    def _fin():
        out_cls_ref[0, 0] = (tot_ref[0, 0] + acc_ref[0]) / float(_NV * _B)
        out_reg_ref[0, 0] = acc_ref[1] / float(_NV * _C * _B)


def _dense_call_kwargs():
    return dict(
        grid=(_B,),
        in_specs=[
            pl.BlockSpec((1, _N, _C), lambda b: (b, 0, 0)),
            pl.BlockSpec((1, 3, _A), lambda b: (b, 0, 0)),
        ],
        out_specs=[
            pl.BlockSpec(memory_space=pltpu.SMEM),
            pl.BlockSpec((1, 1, _NV), lambda b: (b, 0, 0)),
        ],
        out_shape=[
            jax.ShapeDtypeStruct((1, 1), jnp.float32),
            jax.ShapeDtypeStruct((_B, 1, _NV), jnp.float32),
        ],
        scratch_shapes=[pltpu.SMEM((1,), jnp.float32)],
    )


def _final_call_kwargs():
    return dict(
        grid=(_B,),
        in_specs=[
            pl.BlockSpec((1, 3, _A), lambda b: (b, 0, 0)),
            pl.BlockSpec((1, _NV, _C), lambda b: (b, 0, 0)),
            pl.BlockSpec((1, 1, _NV), lambda b: (b, 0, 0)),
            pl.BlockSpec(memory_space=pltpu.SMEM),
        ],
        out_specs=[
            pl.BlockSpec(memory_space=pltpu.SMEM),
            pl.BlockSpec(memory_space=pltpu.SMEM),
        ],
        out_shape=[
            jax.ShapeDtypeStruct((1, 1), jnp.float32),
            jax.ShapeDtypeStruct((1, 1), jnp.float32),
        ],
        scratch_shapes=[pltpu.SMEM((2,), jnp.float32)],
    )


def kernel(classifications, regressions, annotations):
    reg2d = regressions.reshape(_B * _N, _C)
    ann_t = annotations.transpose(0, 2, 1)
    rrows = _sc_gather(ann_t, reg2d)

    tot, cg = pl.pallas_call(_dense_body, **_dense_call_kwargs())(
        classifications, ann_t)
    out_c, out_r = pl.pallas_call(_final_body, **_final_call_kwargs())(
        ann_t, rrows, cg, tot)
    return out_c.reshape(1), out_r.reshape(1)


# speedup vs baseline: 5.0539x; 3.0440x over previous
import functools

import jax
import jax.numpy as jnp
from jax import lax
from jax.experimental import pallas as pl
from jax.experimental.pallas import tpu as pltpu
from jax.experimental.pallas import tpu_sc as plsc

_B, _N, _C, _A = 16, 8192, 80, 64
_NV = 48


def _sc_scatter_counts(ann_t, zeros_n):
    mesh = plsc.VectorSubcoreMesh(core_axis_name="c", subcore_axis_name="s")

    @functools.partial(
        pl.kernel,
        mesh=mesh,
        out_type=jax.ShapeDtypeStruct((_B, 1, _N), jnp.float32),
        scratch_types=[
            pltpu.VMEM((_A,), jnp.float32),
            pltpu.VMEM((_A,), jnp.float32),
            pltpu.VMEM((_N,), jnp.float32),
        ],
        compiler_params=pltpu.CompilerParams(use_tc_tiling_on_sc=False,
                                             needs_layout_passes=False),
    )
    def k(ann_hbm, zeros_hbm, w_hbm, s_v, e_v, w_v):
        cid = lax.axis_index("c")
        sid = lax.axis_index("s")
        wid = sid * 2 + cid

        @pl.when(wid < _B)
        def _():
            b = wid
            pltpu.sync_copy(zeros_hbm, w_v)
            pltpu.sync_copy(ann_hbm.at[b, 0], s_v)
            pltpu.sync_copy(ann_hbm.at[b, 1], e_v)
            ones = jnp.ones((16,), jnp.float32)
            for j in range(_NV // 16):
                sv = s_v[pl.ds(j * 16, 16)]
                ev = e_v[pl.ds(j * 16, 16)]
                tp = ((sv + ev) * 0.5) * 100.0
                plsc.addupdate_scatter(w_v, [tp.astype(jnp.int32)], ones)
            pltpu.sync_copy(w_v, w_hbm.at[b, 0])

    return k(ann_t, zeros_n)


def _smooth_l1(d):
    return jnp.where(d <= 1.0, 0.5 * d * d, d - 0.5)


def _tc_body(ct_ref, rt_ref, ann_ref, w_ref, out_cls_ref, out_reg_ref, acc_ref):
    b = pl.program_id(0)

    @pl.when(b == 0)
    def _init():
        acc_ref[0] = 0.0
        acc_ref[1] = 0.0

    xc = jnp.clip(ct_ref[0], 0.0001, 1.0 - 0.0001)
    part = jnp.sum(xc * xc * jnp.log(1.0 - xc))

    ann = ann_ref[0]
    s = ann[0:1, :]
    e = ann[1:2, :]
    cl = ann[2:3, :]
    pii = (((s + e) * 0.5) * 100.0).astype(jnp.int32)
    pif = pii.astype(jnp.float32)
    clf = cl.astype(jnp.int32).astype(jnp.float32)
    lane = lax.broadcasted_iota(jnp.int32, (1, _A), 1)
    validl = (lane < _NV).astype(jnp.float32)
    keyl = pif * float(_C) + clf

    ident = (lax.broadcasted_iota(jnp.int32, (_A, _A), 0)
             == lax.broadcasted_iota(jnp.int32, (_A, _A), 1)).astype(jnp.float32)

    def to_sub(v):
        return jnp.sum(ident * v, axis=1, keepdims=True)

    key_s = to_sub(keyl)
    pi_s = to_sub(pif)
    valid_s = to_sub(validl)
    js = lax.broadcasted_iota(jnp.int32, (_A, _A), 0)
    ks = lax.broadcasted_iota(jnp.int32, (_A, _A), 1)
    dup = (key_s == keyl) & (js < ks) & (valid_s > 0.0)
    dup_l = jnp.sum(dup.astype(jnp.float32), axis=0, keepdims=True)
    m_l = validl * (dup_l == 0.0).astype(jnp.float32)
    cntm = (pi_s == pif) & (valid_s > 0.0)
    cnt_l = jnp.sum(cntm.astype(jnp.float32), axis=0, keepdims=True)

    m_s = to_sub(m_l)[:_NV, :]
    cnt_s = to_sub(cnt_l)[:_NV, :]
    cl_s = to_sub(clf)[:_NV, :].astype(jnp.int32)
    pi_si = pi_s[:_NV, :].astype(jnp.int32)

    ohc = (lax.broadcasted_iota(jnp.int32, (_NV, _C), 1)
           == cl_s).astype(jnp.float32)
    oh48 = lax.broadcasted_iota(jnp.int32, (_NV, _N), 1) == pi_si

    rowsc = jnp.dot(ohc, xc, preferred_element_type=jnp.float32,
                    precision=lax.Precision.HIGHEST)
    cg = jnp.sum(jnp.where(oh48, rowsc, 0.0), axis=1, keepdims=True)

    rr = rt_ref[0]
    rowsr = jnp.dot(ohc, rr, preferred_element_type=jnp.float32,
                    precision=lax.Precision.HIGHEST)
    g = jnp.sum(jnp.where(oh48, rowsr, 0.0), axis=1, keepdims=True)

    scol = jnp.sum(_smooth_l1(jnp.abs(rr)), axis=0, keepdims=True)
    reg_base = jnp.sum(w_ref[0] * scol)

    l1 = -0.25 * (1.0 - cg) * (1.0 - cg) * jnp.log(cg)
    l0 = -0.75 * cg * cg * jnp.log(1.0 - cg)
    cls_corr = jnp.sum(m_s * (l1 - l0))
    reg_corr = jnp.sum(m_s * cnt_s * (_smooth_l1(jnp.abs(1.0 - g))
                                      - _smooth_l1(jnp.abs(g))))

    acc_ref[0] = acc_ref[0] + (-0.75) * part + cls_corr
    acc_ref[1] = acc_ref[1] + reg_base + reg_corr

    @pl.when(b == _B - 1)
    def _fin():
        out_cls_ref[0, 0] = acc_ref[0] / float(_NV * _B)
        out_reg_ref[0, 0] = acc_ref[1] / float(_NV * _C * _B)


def _tc_call_kwargs():
    return dict(
        grid=(_B,),
        in_specs=[
            pl.BlockSpec((1, _C, _N), lambda b: (b, 0, 0)),
            pl.BlockSpec((1, _C, _N), lambda b: (b, 0, 0)),
            pl.BlockSpec((1, 3, _A), lambda b: (b, 0, 0)),
            pl.BlockSpec((1, 1, _N), lambda b: (b, 0, 0)),
        ],
        out_specs=[
            pl.BlockSpec(memory_space=pltpu.SMEM),
            pl.BlockSpec(memory_space=pltpu.SMEM),
        ],
        out_shape=[
            jax.ShapeDtypeStruct((1, 1), jnp.float32),
            jax.ShapeDtypeStruct((1, 1), jnp.float32),
        ],
        scratch_shapes=[pltpu.SMEM((2,), jnp.float32)],
    )


def kernel(classifications, regressions, annotations):
    ann_t = annotations.transpose(0, 2, 1)
    zeros_n = jnp.zeros((_N,), jnp.float32)
    w = _sc_scatter_counts(ann_t, zeros_n)

    ct = classifications.transpose(0, 2, 1)
    rt = regressions.transpose(0, 2, 1)
    out_c, out_r = pl.pallas_call(_tc_body, **_tc_call_kwargs())(
        ct, rt, ann_t, w)
    return out_c.reshape(1), out_r.reshape(1)
